# baseline (speedup 1.0000x reference)
import functools

import jax
import jax.numpy as jnp
from jax import lax
from jax.experimental import pallas as pl
from jax.experimental.pallas import tpu as pltpu
from jax.experimental.pallas import tpu_sc as plsc

B, S, H = 4, 8192, 1024
M = 1024
NC, NS = 2, 16
NW = NC * NS
RPW = (B * M) // NW
CH = 64
NCH = RPW // CH


def _make_kernel():
  mesh = plsc.VectorSubcoreMesh(core_axis_name="c", subcore_axis_name="s")

  @functools.partial(
      pl.kernel,
      mesh=mesh,
      out_type=jax.ShapeDtypeStruct((B * M // 2, 2 * H), jnp.float32),
      scratch_types=[
          pltpu.VMEM((RPW,), jnp.int32),
          pltpu.VMEM((32, 2 * H), jnp.float32),
          pltpu.SemaphoreType.DMA,
      ],
  )
  def gather_kernel(mp_hbm, x_hbm, out_hbm, idx_v, rows_v, sem):
    wid = lax.axis_index("s") * NC + lax.axis_index("c")
    base = wid * RPW
    b = base // M
    col = base % M
    pltpu.sync_copy(mp_hbm.at[b, pl.ds(col, RPW)], idx_v)
    for i in range(RPW // 16):
      idx_v[pl.ds(i * 16, 16)] = lax.shift_right_logical(
          idx_v[pl.ds(i * 16, 16)], 1)

    def body(c, carry):
      pltpu.async_copy(
          x_hbm.at[idx_v.at[pl.ds(c * 32, 32)]], rows_v, sem).wait()
      return carry

    lax.fori_loop(0, 2, body, 0)
    pltpu.sync_copy(rows_v, out_hbm.at[pl.ds(pl.multiple_of(base // 2, 8), 32)])

  return gather_kernel


_gather = _make_kernel()


@jax.jit
def kernel(x, masked_position):
  out = _gather(masked_position, x.reshape(B * S // 2, 2 * H))
  return out.reshape(B, M, H)


# speedup vs baseline: 5.7051x; 5.7051x over previous
import functools

import jax
import jax.numpy as jnp
from jax import lax
from jax.experimental import pallas as pl
from jax.experimental.pallas import tpu as pltpu
from jax.experimental.pallas import tpu_sc as plsc

B, S, H = 4, 8192, 1024
M = 1024
NC, NS = 2, 16
NW = NC * NS
RPW = (B * M) // NW
CA, CB, CT = 64, 56, 8


def _make_kernel():
  mesh = plsc.VectorSubcoreMesh(core_axis_name="c", subcore_axis_name="s")

  @functools.partial(
      pl.kernel,
      mesh=mesh,
      out_type=jax.ShapeDtypeStruct((B * M, H), jnp.float32),
      scratch_types=[
          pltpu.VMEM((RPW,), jnp.int32),
          pltpu.VMEM((CA, H), jnp.float32),
          pltpu.VMEM((CB, H), jnp.float32),
          pltpu.SemaphoreType.DMA,
          pltpu.SemaphoreType.DMA,
          pltpu.SemaphoreType.DMA,
          pltpu.SemaphoreType.DMA,
      ],
  )
  def gather_kernel(mp_hbm, x_hbm, out_hbm, idx_v, buf_a, buf_b,
                    gsem_a, gsem_b, osem_a, osem_b):
    wid = lax.axis_index("s") * NC + lax.axis_index("c")
    base = wid * RPW
    b = base // M
    col = base % M
    pltpu.sync_copy(mp_hbm.at[b, pl.ds(col, RPW)], idx_v)
    boff = b * S
    for i in range(RPW // 16):
      idx_v[pl.ds(i * 16, 16)] = idx_v[pl.ds(i * 16, 16)] + boff

    g_a = pltpu.async_copy(x_hbm.at[idx_v.at[pl.ds(0, CA)]], buf_a, gsem_a)
    g_b = pltpu.async_copy(x_hbm.at[idx_v.at[pl.ds(CA, CB)]], buf_b, gsem_b)
    g_a.wait()
    s_a = pltpu.async_copy(buf_a, out_hbm.at[pl.ds(base, CA)], osem_a)
    g_b.wait()
    s_b = pltpu.async_copy(buf_b, out_hbm.at[pl.ds(base + CA, CB)], osem_b)
    s_a.wait()
    pltpu.async_copy(x_hbm.at[idx_v.at[pl.ds(CA + CB, CT)]],
                     buf_a.at[pl.ds(0, CT)], gsem_a).wait()
    pltpu.sync_copy(buf_a.at[pl.ds(0, CT)],
                    out_hbm.at[pl.ds(base + CA + CB, CT)])
    s_b.wait()

  return gather_kernel


_gather = _make_kernel()


@jax.jit
def kernel(x, masked_position):
  out = _gather(masked_position, x.reshape(B * S, H))
  return out.reshape(B, M, H)


# speedup vs baseline: 5.7730x; 1.0119x over previous
import functools

import jax
import jax.numpy as jnp
from jax import lax
from jax.experimental import pallas as pl
from jax.experimental.pallas import tpu as pltpu
from jax.experimental.pallas import tpu_sc as plsc

B, S, H = 4, 8192, 1024
M = 1024
NC, NS = 2, 16
NW = NC * NS
RPW = (B * M) // NW
CA, CB, CT = 64, 56, 8


def _make_kernel():
  mesh = plsc.VectorSubcoreMesh(core_axis_name="c", subcore_axis_name="s")

  @functools.partial(
      pl.kernel,
      mesh=mesh,
      out_type=jax.ShapeDtypeStruct((B * M, H), jnp.float32),
      scratch_types=[
          pltpu.VMEM((RPW,), jnp.int32),
          pltpu.VMEM((CA, H), jnp.float32),
          pltpu.VMEM((CB, H), jnp.float32),
          pltpu.SemaphoreType.DMA,
          pltpu.SemaphoreType.DMA,
          pltpu.SemaphoreType.DMA,
          pltpu.SemaphoreType.DMA,
      ],
  )
  def gather_kernel(mp_hbm, x_hbm, out_hbm, idx_v, buf_a, buf_b,
                    gsem_a, gsem_b, osem_a, osem_b):
    wid = lax.axis_index("s") * NC + lax.axis_index("c")
    base = wid * RPW
    b = base // M
    col = base % M
    pltpu.sync_copy(mp_hbm.at[b, pl.ds(col, RPW)], idx_v)
    boff = b * S
    for i in range(RPW // 16):
      idx_v[pl.ds(i * 16, 16)] = idx_v[pl.ds(i * 16, 16)] + boff

    g_a = pltpu.async_copy(x_hbm.at[idx_v.at[pl.ds(0, CA)]], buf_a, gsem_a)
    g_a.wait()
    s_a = pltpu.async_copy(buf_a, out_hbm.at[pl.ds(base, CA)], osem_a)
    g_b = pltpu.async_copy(x_hbm.at[idx_v.at[pl.ds(CA, CB)]], buf_b, gsem_b)
    s_a.wait()
    g_t = pltpu.async_copy(x_hbm.at[idx_v.at[pl.ds(CA + CB, CT)]],
                           buf_a.at[pl.ds(0, CT)], gsem_a)
    g_b.wait()
    s_b = pltpu.async_copy(buf_b, out_hbm.at[pl.ds(base + CA, CB)], osem_b)
    g_t.wait()
    pltpu.sync_copy(buf_a.at[pl.ds(0, CT)],
                    out_hbm.at[pl.ds(base + CA + CB, CT)])
    s_b.wait()

  return gather_kernel


_gather = _make_kernel()


@jax.jit
def kernel(x, masked_position):
  out = _gather(masked_position, x.reshape(B * S, H))
  return out.reshape(B, M, H)
